# baseline (speedup 1.0000x reference)
import functools

import jax
import jax.numpy as jnp
from jax import lax
from jax.experimental import pallas as pl
from jax.experimental.pallas import tpu as pltpu
from jax.experimental.pallas import tpu_sc as plsc

L = 16
ROWS = 64
ROW_LEN = 32768
K = 64
CHUNK = 128
NCHUNKS = ROW_LEN // CHUNK
POOL_CAP = 512
NEG_INF = float("-inf")

def _lane():
    return lax.iota(jnp.int32, L)


def _flip(v):
    return jnp.take(v, (L - 1) - _lane())


def _cx(w, j, want_max):
    p = jnp.take(w, _lane() ^ j)
    return jnp.where(want_max, jnp.maximum(w, p), jnp.minimum(w, p))


def _sort16d(v):
    w = v
    for k in (2, 4, 8, 16):
        j = k // 2
        while j > 0:
            ln = _lane()
            lk, lj = k.bit_length() - 1, j.bit_length() - 1
            want_max = (((ln >> lk) ^ (ln >> lj)) & 1) == 0
            w = _cx(w, j, want_max)
            j //= 2
    return w


def _bm16d(v):
    w = v
    for j in (8, 4, 2, 1):
        w = _cx(w, j, (_lane() & j) == 0)
    return w


def _merge32d(a, b):
    fb = _flip(b)
    return [_bm16d(jnp.maximum(a, fb)), _bm16d(jnp.minimum(a, fb))]


def _clean32(x0, x1):
    return [_bm16d(jnp.maximum(x0, x1)), _bm16d(jnp.minimum(x0, x1))]


def _merge64d(a, b):
    f0 = _flip(b[1])
    f1 = _flip(b[0])
    h = _clean32(jnp.maximum(a[0], f0), jnp.maximum(a[1], f1))
    l = _clean32(jnp.minimum(a[0], f0), jnp.minimum(a[1], f1))
    return h + l


def _top64_of_two(a, b):
    h = [jnp.maximum(a[i], _flip(b[3 - i])) for i in range(4)]
    top = _clean32(jnp.maximum(h[0], h[2]), jnp.maximum(h[1], h[3]))
    bot = _clean32(jnp.minimum(h[0], h[2]), jnp.minimum(h[1], h[3]))
    return top + bot


def _select_top64(vs):
    s16 = [_sort16d(v) for v in vs]
    s32 = [_merge32d(s16[2 * i], s16[2 * i + 1]) for i in range(len(s16) // 2)]
    s64 = [_merge64d(s32[2 * i], s32[2 * i + 1]) for i in range(len(s32) // 2)]
    while len(s64) > 1:
        s64 = [_top64_of_two(s64[2 * i], s64[2 * i + 1])
               for i in range(len(s64) // 2)]
    return s64[0]


def _bfly_max(v):
    w = v
    for j in (1, 2, 4, 8):
        w = jnp.maximum(w, jnp.take(w, _lane() ^ j))
    return w


def _make_sc_kernel():
    mesh = plsc.VectorSubcoreMesh(
        core_axis_name="c", subcore_axis_name="s", num_cores=2, num_subcores=16
    )

    @functools.partial(
        pl.kernel,
        out_type=jax.ShapeDtypeStruct((ROWS * K,), jnp.float32),
        mesh=mesh,
        scratch_types=[
            pltpu.VMEM((ROW_LEN,), jnp.float32),
            pltpu.VMEM((NCHUNKS * L,), jnp.float32),
            pltpu.VMEM((NCHUNKS,), jnp.float32),
            pltpu.VMEM((POOL_CAP,), jnp.float32),
        ],
    )
    def topk_kernel(x_hbm, out_hbm, row_v, cm_splat, cm_c, pool):
        neg = jnp.full((L,), NEG_INF, jnp.float32)
        wid = lax.axis_index("s") * 2 + lax.axis_index("c")

        def rebuild(cnt, t):
            for jj in range(POOL_CAP // L):
                @pl.when(jj * L >= cnt)
                def _():
                    pool[pl.ds(jj * L, L)] = neg
            top = _select_top64(
                [pool[pl.ds(jj * L, L)] for jj in range(POOL_CAP // L)]
            )
            for jj in range(4):
                pool[pl.ds(jj * L, L)] = top[jj]
            return jnp.int32(K), jnp.maximum(t, top[3][15])

        def do_row(r, _):
            row = wid * 2 + r
            pltpu.sync_copy(x_hbm.at[pl.ds(row * ROW_LEN, ROW_LEN)], row_v)

            def p1_body(i, acc):
                for u in range(4):
                    c = i * 4 + u
                    off = c * CHUNK
                    vs = [row_v[pl.ds(off + q * L, L)] for q in range(8)]
                    m01 = jnp.maximum(vs[0], vs[1])
                    m23 = jnp.maximum(vs[2], vs[3])
                    m45 = jnp.maximum(vs[4], vs[5])
                    m67 = jnp.maximum(vs[6], vs[7])
                    lm = jnp.maximum(jnp.maximum(m01, m23),
                                     jnp.maximum(m45, m67))
                    bm = _bfly_max(lm)
                    cm_splat[pl.ds(c * L, L)] = bm
                    acc = jnp.where(_lane() == (c & (L - 1)), bm, acc)
                    cm_c[pl.ds((i // 4) * L, L)] = acc
                return acc

            lax.fori_loop(0, NCHUNKS // 4, p1_body, neg)

            cm_top = _select_top64(
                [cm_c[pl.ds(jj * L, L)] for jj in range(NCHUNKS // L)]
            )
            t0 = cm_top[3][15]

            def p2_body(i, carry):
                cnt, t = carry
                m = cm_splat[pl.ds(i * L, L)][0]

                def hit(cnt, t):
                    cnt, t = lax.cond(
                        cnt > POOL_CAP - CHUNK, rebuild,
                        lambda c, tt: (c, tt), cnt, t,
                    )
                    for q in range(8):
                        v = row_v[pl.ds(i * CHUNK + q * L, L)]
                        vb = _bfly_max(v)

                        def dump(c):
                            pool[pl.ds(c, L)] = jnp.where(v >= t, v, neg)
                            return c + L

                        cnt = lax.cond(vb[0] >= t, dump, lambda c: c, cnt)
                    return cnt, t

                return lax.cond(m >= t, hit, lambda c, tt: (c, tt), cnt, t)

            cnt, _t = lax.fori_loop(
                0, NCHUNKS, p2_body, (jnp.int32(0), t0)
            )

            cnt, _t = rebuild(cnt, _t)
            pltpu.sync_copy(pool.at[pl.ds(0, K)], out_hbm.at[pl.ds(row * K, K)])
            return 0

        lax.fori_loop(0, 2, do_row, 0)

    return topk_kernel


_topk = _make_sc_kernel()


def kernel(x):
    return _topk(x.reshape(-1)).reshape(ROWS, K)


# speedup vs baseline: 1.0481x; 1.0481x over previous
import functools

import jax
import jax.numpy as jnp
from jax import lax
from jax.experimental import pallas as pl
from jax.experimental.pallas import tpu as pltpu
from jax.experimental.pallas import tpu_sc as plsc

L = 16
ROWS = 64
ROW_LEN = 32768
K = 64
CHUNK = 128
NCHUNKS = ROW_LEN // CHUNK
NEG_INF = float("-inf")

def _lane():
    return lax.iota(jnp.int32, L)


def _flip(v):
    return jnp.take(v, (L - 1) - _lane())


def _cx(w, j, want_max):
    p = jnp.take(w, _lane() ^ j)
    return jnp.where(want_max, jnp.maximum(w, p), jnp.minimum(w, p))


def _sort16d(v):
    w = v
    for k in (2, 4, 8, 16):
        j = k // 2
        while j > 0:
            ln = _lane()
            lk, lj = k.bit_length() - 1, j.bit_length() - 1
            want_max = (((ln >> lk) ^ (ln >> lj)) & 1) == 0
            w = _cx(w, j, want_max)
            j //= 2
    return w


def _bm16d(v):
    w = v
    for j in (8, 4, 2, 1):
        w = _cx(w, j, (_lane() & j) == 0)
    return w


def _merge32d(a, b):
    fb = _flip(b)
    return [_bm16d(jnp.maximum(a, fb)), _bm16d(jnp.minimum(a, fb))]


def _clean32(x0, x1):
    return [_bm16d(jnp.maximum(x0, x1)), _bm16d(jnp.minimum(x0, x1))]


def _merge64d(a, b):
    f0 = _flip(b[1])
    f1 = _flip(b[0])
    h = _clean32(jnp.maximum(a[0], f0), jnp.maximum(a[1], f1))
    l = _clean32(jnp.minimum(a[0], f0), jnp.minimum(a[1], f1))
    return h + l


def _top64_of_two(a, b):
    h = [jnp.maximum(a[i], _flip(b[3 - i])) for i in range(4)]
    top = _clean32(jnp.maximum(h[0], h[2]), jnp.maximum(h[1], h[3]))
    bot = _clean32(jnp.minimum(h[0], h[2]), jnp.minimum(h[1], h[3]))
    return top + bot


def _select_top64(vs):
    s16 = [_sort16d(v) for v in vs]
    s32 = [_merge32d(s16[2 * i], s16[2 * i + 1]) for i in range(len(s16) // 2)]
    s64 = [_merge64d(s32[2 * i], s32[2 * i + 1]) for i in range(len(s32) // 2)]
    while len(s64) > 1:
        s64 = [_top64_of_two(s64[2 * i], s64[2 * i + 1])
               for i in range(len(s64) // 2)]
    return s64[0]


def _bfly_max(v):
    w = v
    for j in (1, 2, 4, 8):
        w = jnp.maximum(w, jnp.take(w, _lane() ^ j))
    return w


def _make_sc_kernel():
    mesh = plsc.VectorSubcoreMesh(
        core_axis_name="c", subcore_axis_name="s", num_cores=2, num_subcores=16
    )

    @functools.partial(
        pl.kernel,
        out_type=jax.ShapeDtypeStruct((ROWS * K,), jnp.float32),
        mesh=mesh,
        scratch_types=[
            pltpu.VMEM((ROW_LEN,), jnp.float32),
            pltpu.VMEM((NCHUNKS * L,), jnp.float32),
            pltpu.VMEM((NCHUNKS,), jnp.float32),
            pltpu.VMEM((K,), jnp.float32),
        ],
    )
    def topk_kernel(x_hbm, out_hbm, row_v, cm_splat, cm_c, topbuf):
        neg = jnp.full((L,), NEG_INF, jnp.float32)
        wid = lax.axis_index("s") * 2 + lax.axis_index("c")

        def do_row(r, _):
            row = wid * 2 + r
            pltpu.sync_copy(x_hbm.at[pl.ds(row * ROW_LEN, ROW_LEN)], row_v)

            def p1_body(i, acc):
                for u in range(4):
                    c = i * 4 + u
                    off = c * CHUNK
                    vs = [row_v[pl.ds(off + q * L, L)] for q in range(8)]
                    m01 = jnp.maximum(vs[0], vs[1])
                    m23 = jnp.maximum(vs[2], vs[3])
                    m45 = jnp.maximum(vs[4], vs[5])
                    m67 = jnp.maximum(vs[6], vs[7])
                    lm = jnp.maximum(jnp.maximum(m01, m23),
                                     jnp.maximum(m45, m67))
                    bm = _bfly_max(lm)
                    cm_splat[pl.ds(c * L, L)] = bm
                    acc = jnp.where(_lane() == (c & (L - 1)), bm, acc)
                    cm_c[pl.ds((i // 4) * L, L)] = acc
                return acc

            lax.fori_loop(0, NCHUNKS // 4, p1_body, neg)

            cm_top = _select_top64(
                [cm_c[pl.ds(jj * L, L)] for jj in range(NCHUNKS // L)]
            )
            t0 = cm_top[3][15]

            for jj in range(4):
                topbuf[pl.ds(jj * L, L)] = neg

            def p2_body(i, t):
                m = cm_splat[pl.ds(i * L, L)][0]

                def hit(t):
                    def q_body(q, tt):
                        v = row_v[pl.ds(i * CHUNK + q * L, L)]
                        vb = _bfly_max(v)

                        def ins(ttt):
                            s = _sort16d(v)
                            t0v = topbuf[pl.ds(0, L)]
                            t1v = topbuf[pl.ds(L, L)]
                            t2v = topbuf[pl.ds(2 * L, L)]
                            t3v = topbuf[pl.ds(3 * L, L)]
                            h3 = jnp.maximum(t3v, _flip(s))
                            top = _clean32(t0v, jnp.maximum(t1v, h3))
                            bot = _clean32(t2v, jnp.minimum(t1v, h3))
                            topbuf[pl.ds(0, L)] = top[0]
                            topbuf[pl.ds(L, L)] = top[1]
                            topbuf[pl.ds(2 * L, L)] = bot[0]
                            topbuf[pl.ds(3 * L, L)] = bot[1]
                            return jnp.maximum(ttt, bot[1][15])

                        return lax.cond(vb[0] >= tt, ins, lambda s_: s_, tt)

                    return lax.fori_loop(0, 8, q_body, t)

                return lax.cond(m >= t, hit, lambda tt: tt, t)

            lax.fori_loop(0, NCHUNKS, p2_body, t0)

            pltpu.sync_copy(topbuf.at[pl.ds(0, K)],
                            out_hbm.at[pl.ds(row * K, K)])
            return 0

        lax.fori_loop(0, 2, do_row, 0)

    return topk_kernel


_topk = _make_sc_kernel()


def kernel(x):
    return _topk(x.reshape(-1)).reshape(ROWS, K)


# speedup vs baseline: 1.0543x; 1.0059x over previous
import functools

import jax
import jax.numpy as jnp
from jax import lax
from jax.experimental import pallas as pl
from jax.experimental.pallas import tpu as pltpu
from jax.experimental.pallas import tpu_sc as plsc

L = 16
ROWS = 64
ROW_LEN = 32768
K = 64
CHUNK = 128
NCHUNKS = ROW_LEN // CHUNK
NEG_INF = float("-inf")

def _lane():
    return lax.iota(jnp.int32, L)


def _flip(v):
    return jnp.take(v, (L - 1) - _lane())


def _cx(w, j, want_max):
    p = jnp.take(w, _lane() ^ j)
    return jnp.where(want_max, jnp.maximum(w, p), jnp.minimum(w, p))


def _sort16d(v):
    w = v
    for k in (2, 4, 8, 16):
        j = k // 2
        while j > 0:
            ln = _lane()
            lk, lj = k.bit_length() - 1, j.bit_length() - 1
            want_max = (((ln >> lk) ^ (ln >> lj)) & 1) == 0
            w = _cx(w, j, want_max)
            j //= 2
    return w


def _bm16d(v):
    w = v
    for j in (8, 4, 2, 1):
        w = _cx(w, j, (_lane() & j) == 0)
    return w


def _merge32d(a, b):
    fb = _flip(b)
    return [_bm16d(jnp.maximum(a, fb)), _bm16d(jnp.minimum(a, fb))]


def _clean32(x0, x1):
    return [_bm16d(jnp.maximum(x0, x1)), _bm16d(jnp.minimum(x0, x1))]


def _merge64d(a, b):
    f0 = _flip(b[1])
    f1 = _flip(b[0])
    h = _clean32(jnp.maximum(a[0], f0), jnp.maximum(a[1], f1))
    l = _clean32(jnp.minimum(a[0], f0), jnp.minimum(a[1], f1))
    return h + l


def _top64_of_two(a, b):
    h = [jnp.maximum(a[i], _flip(b[3 - i])) for i in range(4)]
    top = _clean32(jnp.maximum(h[0], h[2]), jnp.maximum(h[1], h[3]))
    bot = _clean32(jnp.minimum(h[0], h[2]), jnp.minimum(h[1], h[3]))
    return top + bot


def _select_top64(vs):
    s16 = [_sort16d(v) for v in vs]
    s32 = [_merge32d(s16[2 * i], s16[2 * i + 1]) for i in range(len(s16) // 2)]
    s64 = [_merge64d(s32[2 * i], s32[2 * i + 1]) for i in range(len(s32) // 2)]
    while len(s64) > 1:
        s64 = [_top64_of_two(s64[2 * i], s64[2 * i + 1])
               for i in range(len(s64) // 2)]
    return s64[0]


def _bfly_max(v):
    w = v
    for j in (1, 2, 4, 8):
        w = jnp.maximum(w, jnp.take(w, _lane() ^ j))
    return w


def _make_sc_kernel():
    mesh = plsc.VectorSubcoreMesh(
        core_axis_name="c", subcore_axis_name="s", num_cores=2, num_subcores=16
    )

    @functools.partial(
        pl.kernel,
        out_type=jax.ShapeDtypeStruct((ROWS * K,), jnp.float32),
        mesh=mesh,
        scratch_types=[
            pltpu.VMEM((2 * ROW_LEN,), jnp.float32),
            pltpu.VMEM((NCHUNKS * L,), jnp.float32),
            pltpu.VMEM((NCHUNKS,), jnp.float32),
            pltpu.VMEM((K,), jnp.float32),
            pltpu.SemaphoreType.DMA,
            pltpu.SemaphoreType.DMA,
        ],
    )
    def topk_kernel(x_hbm, out_hbm, row_v, cm_splat, cm_c, topbuf,
                    sem_a, sem_b):
        neg = jnp.full((L,), NEG_INF, jnp.float32)
        wid = lax.axis_index("s") * 2 + lax.axis_index("c")
        r0 = wid * 2

        cp_a = pltpu.async_copy(
            x_hbm.at[pl.ds(r0 * ROW_LEN, ROW_LEN)],
            row_v.at[pl.ds(0, ROW_LEN)], sem_a)
        cp_b = pltpu.async_copy(
            x_hbm.at[pl.ds((r0 + 1) * ROW_LEN, ROW_LEN)],
            row_v.at[pl.ds(ROW_LEN, ROW_LEN)], sem_b)

        def do_row(r, _):
            row = r0 + r
            base = r * ROW_LEN

            @pl.when(r == 0)
            def _():
                cp_a.wait()

            @pl.when(r != 0)
            def _():
                cp_b.wait()

            def p1_body(i, acc):
                for u in range(4):
                    c = i * 4 + u
                    off = base + c * CHUNK
                    vs = [row_v[pl.ds(off + q * L, L)] for q in range(8)]
                    m01 = jnp.maximum(vs[0], vs[1])
                    m23 = jnp.maximum(vs[2], vs[3])
                    m45 = jnp.maximum(vs[4], vs[5])
                    m67 = jnp.maximum(vs[6], vs[7])
                    lm = jnp.maximum(jnp.maximum(m01, m23),
                                     jnp.maximum(m45, m67))
                    bm = _bfly_max(lm)
                    cm_splat[pl.ds(c * L, L)] = bm
                    acc = jnp.where(_lane() == (c & (L - 1)), bm, acc)
                    cm_c[pl.ds((i // 4) * L, L)] = acc
                return acc

            lax.fori_loop(0, NCHUNKS // 4, p1_body, neg)

            cm_top = _select_top64(
                [cm_c[pl.ds(jj * L, L)] for jj in range(NCHUNKS // L)]
            )
            t0 = cm_top[3][15]

            for jj in range(4):
                topbuf[pl.ds(jj * L, L)] = neg

            def p2_body(i, t):
                m = cm_splat[pl.ds(i * L, L)][0]

                def hit(t):
                    def q_body(q, tt):
                        v = row_v[pl.ds(base + i * CHUNK + q * L, L)]
                        vb = _bfly_max(v)

                        def ins(ttt):
                            s = _sort16d(v)
                            t0v = topbuf[pl.ds(0, L)]
                            t1v = topbuf[pl.ds(L, L)]
                            t2v = topbuf[pl.ds(2 * L, L)]
                            t3v = topbuf[pl.ds(3 * L, L)]
                            h3 = jnp.maximum(t3v, _flip(s))
                            top = _clean32(t0v, jnp.maximum(t1v, h3))
                            bot = _clean32(t2v, jnp.minimum(t1v, h3))
                            topbuf[pl.ds(0, L)] = top[0]
                            topbuf[pl.ds(L, L)] = top[1]
                            topbuf[pl.ds(2 * L, L)] = bot[0]
                            topbuf[pl.ds(3 * L, L)] = bot[1]
                            return jnp.maximum(ttt, bot[1][15])

                        return lax.cond(vb[0] >= tt, ins, lambda s_: s_, tt)

                    return lax.fori_loop(0, 8, q_body, t)

                return lax.cond(m >= t, hit, lambda tt: tt, t)

            lax.fori_loop(0, NCHUNKS, p2_body, t0)

            pltpu.sync_copy(topbuf.at[pl.ds(0, K)],
                            out_hbm.at[pl.ds(row * K, K)])
            return 0

        lax.fori_loop(0, 2, do_row, 0)

    return topk_kernel


_topk = _make_sc_kernel()


def kernel(x):
    return _topk(x.reshape(-1)).reshape(ROWS, K)
